# Initial kernel scaffold; baseline (speedup 1.0000x reference)
import jax
import jax.numpy as jnp
from jax.experimental import pallas as pl


def kernel(confidences, regressions, anchors):
    raise NotImplementedError("write your pallas kernel here")


# speedup vs baseline: 7.0914x; 7.0914x over previous
import jax
import jax.numpy as jnp
from jax.experimental import pallas as pl
from jax.experimental.pallas import tpu as pltpu

_NUM_CLASSES = 90
_TOP_K = 200
_CONF = 0.05
_NMS_T = 0.5
_CROP = 300.0
_NEG = -1e9
_LANES = 128


def _score_cls_kernel(conf_ref, score_ref, cls_ref):
    c = conf_ref[...]
    m = jnp.max(c, axis=1, keepdims=True)
    iota = jax.lax.broadcasted_iota(jnp.int32, c.shape, 1)
    idx = jnp.min(jnp.where(c == m, iota, 2147483647), axis=1, keepdims=True)
    score_ref[...] = m
    cls_ref[...] = idx.astype(jnp.float32)


def _nms_kernel(score_ref, cls_ref, reg_ref, anch_ref, out_ref, clsout_ref,
                bx1_ref, by1_ref, bx2_ref, by2_ref,
                ox1_ref, oy1_ref, ox2_ref, oy2_ref, area_ref):
    R = score_ref.shape[1]
    scores_raw = score_ref[0]
    clsf = cls_ref[0]
    ay1 = anch_ref[0]
    ax1 = anch_ref[1]
    ay2 = anch_ref[2]
    ax2 = anch_ref[3]
    dy = reg_ref[0, 0]
    dx = reg_ref[0, 1]
    dh = reg_ref[0, 2]
    dw = reg_ref[0, 3]
    yc_a = (ay1 + ay2) / 2.0
    xc_a = (ax1 + ax2) / 2.0
    ha = ay2 - ay1
    wa = ax2 - ax1
    w = jnp.exp(dw) * wa
    h = jnp.exp(dh) * ha
    yc = dy * ha + yc_a
    xc = dx * wa + xc_a
    bx1 = jnp.clip(xc - w / 2.0, 0.0, _CROP) / _CROP
    by1 = jnp.clip(yc - h / 2.0, 0.0, _CROP) / _CROP
    bx2 = jnp.clip(xc + w / 2.0, 0.0, _CROP) / _CROP
    by2 = jnp.clip(yc + h / 2.0, 0.0, _CROP) / _CROP
    off = clsf * 2.0
    ox1 = bx1 + off
    oy1 = by1 + off
    ox2 = bx2 + off
    oy2 = by2 + off
    area = jnp.clip(ox2 - ox1, 0.0, None) * jnp.clip(oy2 - oy1, 0.0, None)
    bx1_ref[...] = bx1
    by1_ref[...] = by1
    bx2_ref[...] = bx2
    by2_ref[...] = by2
    ox1_ref[...] = ox1
    oy1_ref[...] = oy1
    ox2_ref[...] = ox2
    oy2_ref[...] = oy2
    area_ref[...] = area

    ridx = jax.lax.broadcasted_iota(jnp.int32, (R, _LANES), 0)
    lidx = jax.lax.broadcasted_iota(jnp.int32, (R, _LANES), 1)
    nidx = ridx * _LANES + lidx
    lane1 = jax.lax.broadcasted_iota(jnp.int32, (1, _LANES), 1)
    s0 = jnp.where(scores_raw > _CONF, scores_raw, _NEG)

    def gather(ref, ri, lmask):
        row = ref[pl.ds(ri, 1), :]
        return jnp.sum(jnp.where(lmask, row, 0.0))

    def body(t, s):
        m = jnp.max(s)
        valid = m > _NEG / 2.0
        sel = jnp.min(jnp.where(s == m, nidx, 2147483647))
        ri = sel // _LANES
        li = sel - ri * _LANES
        lmask = lane1 == li
        sx1 = gather(ox1_ref, ri, lmask)
        sy1 = gather(oy1_ref, ri, lmask)
        sx2 = gather(ox2_ref, ri, lmask)
        sy2 = gather(oy2_ref, ri, lmask)
        sarea = gather(area_ref, ri, lmask)
        dx1 = gather(bx1_ref, ri, lmask)
        dy1 = gather(by1_ref, ri, lmask)
        dx2 = gather(bx2_ref, ri, lmask)
        dy2 = gather(by2_ref, ri, lmask)
        srow = score_ref[0, pl.ds(ri, 1), :]
        sscore = jnp.sum(jnp.where(lmask, srow, 0.0))
        crow = cls_ref[0, pl.ds(ri, 1), :]
        scls = jnp.sum(jnp.where(lmask, crow, 0.0))
        xx1 = jnp.maximum(sx1, ox1)
        yy1 = jnp.maximum(sy1, oy1)
        xx2 = jnp.minimum(sx2, ox2)
        yy2 = jnp.minimum(sy2, oy2)
        inter = jnp.clip(xx2 - xx1, 0.0, None) * jnp.clip(yy2 - yy1, 0.0, None)
        iou = inter / (sarea + area - inter + 1e-8)
        kill = (iou > _NMS_T) | (nidx == sel)
        s_new = jnp.where(kill, _NEG, s)
        v = jnp.where(valid, 1.0, 0.0)
        row = jnp.where(lane1 == 0, dx1, 0.0)
        row = jnp.where(lane1 == 1, dy1, row)
        row = jnp.where(lane1 == 2, dx2, row)
        row = jnp.where(lane1 == 3, dy2, row)
        row = jnp.where(lane1 == 4, sscore, row)
        out_ref[0, pl.ds(t, 1), :] = row * v
        cv = jnp.where(valid, scls, -1.0)
        clsout_ref[0, pl.ds(t, 1), :] = jnp.broadcast_to(cv, (1, _LANES))
        return s_new

    jax.lax.fori_loop(0, _TOP_K, body, s0)


@jax.jit
def kernel(confidences, regressions, anchors):
    B, N, C = confidences.shape
    R = (N + _LANES - 1) // _LANES
    NP = R * _LANES
    pad = NP - N
    conf_p = jnp.pad(confidences, ((0, 0), (0, pad), (0, 0)))
    reg_p = jnp.pad(regressions, ((0, 0), (0, pad), (0, 0)))
    anch_p = jnp.pad(anchors, ((0, pad), (0, 0)))

    rows_total = B * NP
    blk = 1024
    while rows_total % blk != 0:
        blk //= 2
    conf2 = conf_p.reshape(rows_total, C)
    sc, cl = pl.pallas_call(
        _score_cls_kernel,
        grid=(rows_total // blk,),
        in_specs=[pl.BlockSpec((blk, C), lambda i: (i, 0))],
        out_specs=[pl.BlockSpec((blk, 1), lambda i: (i, 0)),
                   pl.BlockSpec((blk, 1), lambda i: (i, 0))],
        out_shape=[jax.ShapeDtypeStruct((rows_total, 1), jnp.float32),
                   jax.ShapeDtypeStruct((rows_total, 1), jnp.float32)],
    )(conf2)
    scores = sc.reshape(B, R, _LANES)
    clsf = cl.reshape(B, R, _LANES)
    reg_t = reg_p.transpose(0, 2, 1).reshape(B, 4, R, _LANES)
    anch_t = anch_p.T.reshape(4, R, _LANES)

    out_p, clsout_p = pl.pallas_call(
        _nms_kernel,
        grid=(B,),
        in_specs=[
            pl.BlockSpec((1, R, _LANES), lambda b: (b, 0, 0)),
            pl.BlockSpec((1, R, _LANES), lambda b: (b, 0, 0)),
            pl.BlockSpec((1, 4, R, _LANES), lambda b: (b, 0, 0, 0)),
            pl.BlockSpec((4, R, _LANES), lambda b: (0, 0, 0)),
        ],
        out_specs=[pl.BlockSpec((1, _TOP_K, _LANES), lambda b: (b, 0, 0)),
                   pl.BlockSpec((1, _TOP_K, _LANES), lambda b: (b, 0, 0))],
        out_shape=[jax.ShapeDtypeStruct((B, _TOP_K, _LANES), jnp.float32),
                   jax.ShapeDtypeStruct((B, _TOP_K, _LANES), jnp.float32)],
        scratch_shapes=[pltpu.VMEM((R, _LANES), jnp.float32)] * 9,
    )(scores, clsf, reg_t, anch_t)
    out = out_p[:, :, :5]
    out_classes = clsout_p[:, :, 0].astype(jnp.int32)
    return out, out_classes
